# baseline (speedup 1.0000x reference)
import jax
import jax.numpy as jnp
from jax import lax
from jax.experimental import pallas as pl
from jax.experimental.pallas import tpu as pltpu
from jax.experimental.pallas import tpu_sc as plsc

B = 4096
DENSE = 64
HIST = 50
N_EMB = 4
EMB_DIM = 64
XCOLS = DENSE + N_EMB * HIST

NC, NS = 2, 16
NW = NC * NS
ROWS_PER_W = B // NW
RCHUNK = 64
NCHUNK = ROWS_PER_W // RCHUNK
NPAIR = RCHUNK // 2
NBUF = 8


def _make_body(t):
    with_dense = t == 0
    ocols = DENSE + EMB_DIM if with_dense else EMB_DIM
    ecol = DENSE if with_dense else 0
    raw0 = 0 if with_dense else DENSE + t * HIST
    xcol0 = (raw0 // 8) * 8
    icol = (DENSE if with_dense else 0) + (raw0 - xcol0)
    xcols = ((icol + HIST + 7) // 8) * 8

    def body(
        x_hbm, w, out_hbm, xv, idxv,
        gb0, gb1, gb2, gb3, gb4, gb5, gb6, gb7, ov,
        s0, s1, s2, s3, s4, s5, s6, s7,
    ):
        gbufs = (gb0, gb1, gb2, gb3, gb4, gb5, gb6, gb7)
        sems = (s0, s1, s2, s3, s4, s5, s6, s7)
        wid = lax.axis_index("c") * NS + lax.axis_index("s")

        def row_step(gbuf, row, sums, cnt):
            g0 = gbuf[row, pl.ds(0, 16)]
            g1 = gbuf[row, pl.ds(16, 16)]
            g2 = gbuf[row, pl.ds(32, 16)]
            g3 = gbuf[row, pl.ds(48, 16)]
            mag = jnp.float32(0)
            for g in (g0, g1, g2, g3):
                a = plsc.bitcast(g, jnp.int32) & jnp.int32(0x7FFFFFFF)
                mag = a if g is g0 else jnp.minimum(mag, a)
            pcnt = plsc.all_reduce_population_count(mag > 0)
            cnt = cnt + jnp.where(pcnt == 16, 1.0, 0.0)
            return (sums[0] + g0, sums[1] + g1, sums[2] + g2, sums[3] + g3), cnt

        def write_row(i, sums, cnt):
            div = jnp.where(cnt == 0.0, jnp.float32(1e-8), cnt)
            for c in range(4):
                ov[i, pl.ds(ecol + 16 * c, 16)] = sums[c] / div

        def reduce_pair(gbuf, p):
            def red(j, carry):
                sa, ca, sb, cb = carry
                for jj in (2 * j, 2 * j + 1):
                    sa, ca = row_step(gbuf, jj, sa, ca)
                    sb, cb = row_step(gbuf, HIST + jj, sb, cb)
                return (sa, ca, sb, cb)

            zero = jnp.zeros((16,), jnp.float32)
            z4 = (zero, zero, zero, zero)
            sa, ca, sb, cb = lax.fori_loop(0, HIST // 2, red, (z4, zero, z4, zero))
            write_row(2 * p, sa, ca)
            write_row(2 * p + 1, sb, cb)

        def chunk_body(chunk, _):
            base = wid * ROWS_PER_W + chunk * RCHUNK
            pltpu.sync_copy(
                x_hbm.at[pl.ds(base, RCHUNK), pl.ds(xcol0, xcols)], xv
            )

            for i in range(RCHUNK):
                if with_dense:
                    for c in range(DENSE // 16):
                        ov[i, pl.ds(16 * c, 16)] = xv[i, pl.ds(16 * c, 16)]
                half = (i % 2) * HIST
                for off in (0, 16, 32, HIST - 16):
                    idxv[i // 2, 0, pl.ds(half + off, 16)] = xv[
                        i, pl.ds(icol + off, 16)
                    ].astype(jnp.int32)

            def issue(k):
                return pltpu.async_copy(
                    w.at[idxv.at[k, 0]], gbufs[k % NBUF], sems[k % NBUF]
                )

            handles = {k: issue(k) for k in range(NBUF - 1)}
            for k in range(NPAIR):
                if k + NBUF - 1 < NPAIR:
                    handles[k + NBUF - 1] = issue(k + NBUF - 1)
                handles[k].wait()
                reduce_pair(gbufs[k % NBUF], k)

            pltpu.sync_copy(ov, out_hbm.at[pl.ds(base, RCHUNK), :])
            return ()

        lax.fori_loop(0, NCHUNK, chunk_body, ())

    mesh = plsc.VectorSubcoreMesh(core_axis_name="c", subcore_axis_name="s")
    return pl.kernel(
        body,
        out_type=jax.ShapeDtypeStruct((B, ocols), jnp.float32),
        mesh=mesh,
        compiler_params=pltpu.CompilerParams(
            needs_layout_passes=False, use_tc_tiling_on_sc=False
        ),
        scratch_types=[
            pltpu.VMEM((RCHUNK, xcols), jnp.float32),
            pltpu.VMEM((NPAIR, 1, 2 * HIST), jnp.int32),
        ]
        + [pltpu.VMEM((2 * HIST, EMB_DIM), jnp.float32)] * NBUF
        + [pltpu.VMEM((RCHUNK, ocols), jnp.float32)]
        + [pltpu.SemaphoreType.DMA] * NBUF,
        name=f"emb_pool_t{t}",
    )


_CALLS = [_make_body(t) for t in range(N_EMB)]


@jax.jit
def kernel(x, W0, W1, W2, W3):
    ws = (W0, W1, W2, W3)
    parts = [None] * N_EMB
    for t in (1, 2, 3, 0):
        parts[t] = _CALLS[t](x, ws[t])
    return jnp.concatenate(parts, axis=1)
